# baseline (speedup 1.0000x reference)
import functools

import jax
import jax.numpy as jnp
from jax import lax
from jax.experimental import pallas as pl
from jax.experimental.pallas import tpu as pltpu
from jax.experimental.pallas import tpu_sc as plsc


def _logit_table_body(w_ref, b_ref, t_ref, s_ref):
    s_ref[...] = jnp.sum(t_ref[...] * w_ref[...], axis=0) + b_ref[0, 0]


def _padded_vocab(V, NS=16):
    return ((V + NS * 1024 - 1) // (NS * 1024)) * (NS * 1024)


def _make_logit_table(V2, D, blk):
    grid = (V2 + blk - 1) // blk
    return pl.pallas_call(
        _logit_table_body,
        grid=(grid,),
        in_specs=[
            pl.BlockSpec((D, 1), lambda i: (0, 0)),
            pl.BlockSpec((1, 1), lambda i: (0, 0)),
            pl.BlockSpec((D, blk), lambda i: (0, i)),
        ],
        out_specs=pl.BlockSpec((blk,), lambda i: (i,)),
        out_shape=jax.ShapeDtypeStruct((V2,), jnp.float32),
    )


def _make_sc_pool(V, B, L):
    info = plsc.get_sparse_core_info()
    NC, NS, LN = info.num_cores, info.num_subcores, info.num_lanes
    NW = NC * NS
    RPW = B // NW
    IPW = RPW * L
    V2 = _padded_vocab(V, NS)
    CH = V2 // NS
    GRP = RPW // LN
    mesh = plsc.VectorSubcoreMesh(core_axis_name="c", subcore_axis_name="s")

    @functools.partial(
        pl.kernel,
        mesh=mesh,
        compiler_params=pltpu.CompilerParams(needs_layout_passes=False),
        out_type=jax.ShapeDtypeStruct((B,), jnp.float32),
        scratch_types=[
            pltpu.VMEM_SHARED((V2,), jnp.float32),
            pltpu.VMEM((IPW,), jnp.int32),
            pltpu.VMEM((IPW,), jnp.float32),
            pltpu.VMEM((RPW,), jnp.float32),
            pltpu.SemaphoreType.DMA,
            pltpu.SemaphoreType.DMA,
        ],
    )
    def sc_pool(s_hbm, xt_hbm, out_hbm, s_sh, idx_v, vals_v, out_v,
                sem, sem_s):
        cid = lax.axis_index("c")
        sid = lax.axis_index("s")
        wid = sid * NC + cid
        base_row = pl.multiple_of(wid * RPW, 8)

        def ix_body(l, c):
            pltpu.make_async_copy(
                xt_hbm.at[l, pl.ds(base_row, RPW)],
                idx_v.at[pl.ds(pl.multiple_of(l * RPW, 8), RPW)],
                sem).start()
            return c

        lax.fori_loop(0, L, ix_body, 0)

        off = pl.multiple_of(sid * CH, 1024)
        pltpu.make_async_copy(s_hbm.at[pl.ds(off, CH)],
                              s_sh.at[pl.ds(off, CH)], sem_s).start()
        pltpu.make_async_copy(s_hbm.at[pl.ds(off, CH)],
                              s_sh.at[pl.ds(off, CH)], sem_s).wait()

        def ix_drain(l, c):
            pltpu.make_async_copy(
                xt_hbm.at[l, pl.ds(base_row, RPW)],
                idx_v.at[pl.ds(pl.multiple_of(l * RPW, 8), RPW)],
                sem).wait()
            return c

        lax.fori_loop(0, L, ix_drain, 0)
        plsc.subcore_barrier()

        pltpu.make_async_copy(s_sh.at[idx_v], vals_v, sem).start()
        pltpu.make_async_copy(s_sh.at[idx_v], vals_v, sem).wait()

        def grp_body(g, c):
            g16 = pl.multiple_of(g * LN, 8)
            acc = jnp.zeros((LN,), jnp.float32)
            for l in range(L):
                acc = acc + vals_v[pl.ds(g16 + l * RPW, LN)]
            z = acc / jnp.float32(L)
            sig = 1.0 / (1.0 + jnp.exp(-z))
            t = sig * 10000.0
            r = (t + 8388608.0) - 8388608.0
            out_v[pl.ds(g * LN, LN)] = r / 10000.0
            return c

        lax.fori_loop(0, GRP, grp_body, 0)

        pltpu.sync_copy(out_v, out_hbm.at[pl.ds(pl.multiple_of(wid * RPW, 8), RPW)])

    return sc_pool


def kernel(x, table, W, b):
    V, D = table.shape
    B, L = x.shape
    s = _make_logit_table(_padded_vocab(V), D, 262144)(
        W.reshape(D, 1), b.reshape(1, 1), table.T)
    xt = x.astype(jnp.int32).T
    out = _make_sc_pool(V, B, L)(s, xt)
    return out.reshape(B, 1)
